# Initial kernel scaffold; baseline (speedup 1.0000x reference)
import jax
import jax.numpy as jnp
from jax.experimental import pallas as pl


def kernel(users, items, Gu, Gi, W1, b1, W2, b2, ui):
    raise NotImplementedError("write your pallas kernel here")


# speedup vs baseline: 3.1481x; 3.1481x over previous
import functools

import jax
import jax.numpy as jnp
from jax import lax
from jax.experimental import pallas as pl
from jax.experimental.pallas import tpu as pltpu
from jax.experimental.pallas import tpu_sc as plsc

NUM_USERS = 100000
NUM_ITEMS = 1000000
K = 16
H = 20
B = 4096

NC = 2
NS = 16
NW = NC * NS
CH = B // NW


def _sc_body(users_h, items_h, gu_tab, gi_tab, ui1_h,
             gu_o, gihist_o, gmi_o,
             users_v, items_v, idx20, idxj, histcol, rows_v, guv, giv, sem):
  wid = lax.axis_index("s") * NC + lax.axis_index("c")
  base = wid * CH

  pltpu.sync_copy(users_h.at[pl.ds(base, CH)], users_v)
  pltpu.sync_copy(items_h.at[pl.ds(base, CH)], items_v)

  pltpu.async_copy(gu_tab.at[users_v], guv, sem).wait()
  pltpu.sync_copy(guv, gu_o.at[pl.ds(base, CH)])
  pltpu.async_copy(gi_tab.at[items_v], giv, sem).wait()
  pltpu.sync_copy(giv, gmi_o.at[pl.ds(base, CH)])

  def mul_body(t, carry):
    sl = pl.ds(t * 16, 16)
    idx20[sl] = users_v[sl] * H
    return carry
  lax.fori_loop(0, CH // 16, mul_body, 0, unroll=True)

  def j_body(j, carry):
    def add_body(t, c):
      sl = pl.ds(t * 16, 16)
      idxj[sl] = idx20[sl] + j
      return c
    lax.fori_loop(0, CH // 16, add_body, 0, unroll=True)
    pltpu.async_copy(ui1_h.at[idxj], histcol, sem).wait()
    pltpu.async_copy(gi_tab.at[histcol], rows_v, sem).wait()
    pltpu.sync_copy(rows_v, gihist_o.at[j, pl.ds(base, CH)])
    return carry
  lax.fori_loop(0, H, j_body, 0)


@functools.partial(jax.jit)
def _sc_gather(users, items, Gu, Gi, ui1):
  mesh = plsc.VectorSubcoreMesh(core_axis_name="c", subcore_axis_name="s")
  f = pl.kernel(
      _sc_body,
      out_type=[
          jax.ShapeDtypeStruct((B, K), jnp.float32),
          jax.ShapeDtypeStruct((H, B, K), jnp.float32),
          jax.ShapeDtypeStruct((B, K), jnp.float32),
      ],
      mesh=mesh,
      scratch_types=[
          pltpu.VMEM((CH,), jnp.int32),
          pltpu.VMEM((CH,), jnp.int32),
          pltpu.VMEM((CH,), jnp.int32),
          pltpu.VMEM((CH,), jnp.int32),
          pltpu.VMEM((CH,), jnp.int32),
          pltpu.VMEM((CH, K), jnp.float32),
          pltpu.VMEM((CH, K), jnp.float32),
          pltpu.VMEM((CH, K), jnp.float32),
          pltpu.SemaphoreType.DMA,
      ],
      compiler_params=pltpu.CompilerParams(use_tc_tiling_on_sc=False),
  )
  return f(users, items, Gu, Gi, ui1)


_T = 512


def _leaky(x):
  return jnp.where(x >= 0, x, 0.01 * x)


def _tc_body(gu_ref, gh_ref, gmi_ref, wa_ref, wb_ref, w2_ref, b1_ref, b2_ref,
             xui_ref, gs_ref):
  gu = gu_ref[...]
  wa = wa_ref[...]
  wb = wb_ref[...]
  w2 = w2_ref[...]
  b1 = b1_ref[...]
  b2 = b2_ref[...]

  au = jnp.dot(gu, wa, preferred_element_type=jnp.float32)

  gh = gh_ref[...].reshape(H * _T, K)
  hi = jnp.dot(gh, wb, preferred_element_type=jnp.float32)
  h1 = hi.reshape(H, _T, K) + (au + b1)[None]
  hbar = jnp.mean(_leaky(h1), axis=0)
  gs = jnp.dot(hbar, w2, preferred_element_type=jnp.float32) + b2

  ai = jnp.dot(gmi_ref[...], wb, preferred_element_type=jnp.float32)
  gl = _leaky(au + ai + b1)
  gui = jnp.dot(gl, w2, preferred_element_type=jnp.float32) + b2

  gs_ref[...] = gs
  xui_ref[...] = jnp.sum(gs * gui, axis=1, keepdims=True)


def _tc_mlp(gu, gihist, gmi, wa, wb, w2, b1, b2):
  grid = B // _T
  wspec = pl.BlockSpec((K, K), lambda i: (0, 0))
  bspec = pl.BlockSpec((1, K), lambda i: (0, 0))
  return pl.pallas_call(
      _tc_body,
      grid=(grid,),
      in_specs=[
          pl.BlockSpec((_T, K), lambda i: (i, 0)),
          pl.BlockSpec((H, _T, K), lambda i: (0, i, 0)),
          pl.BlockSpec((_T, K), lambda i: (i, 0)),
          wspec, wspec, wspec, bspec, bspec,
      ],
      out_specs=[
          pl.BlockSpec((_T, 1), lambda i: (i, 0)),
          pl.BlockSpec((_T, K), lambda i: (i, 0)),
      ],
      out_shape=[
          jax.ShapeDtypeStruct((B, 1), jnp.float32),
          jax.ShapeDtypeStruct((B, K), jnp.float32),
      ],
  )(gu, gihist, gmi, wa, wb, w2, b1, b2)


def kernel(users, items, Gu, Gi, W1, b1, W2, b2, ui):
  ui1 = ui[1]
  w1t = W1.T
  wa = w1t[:K]
  wb = w1t[K:]
  w2t = W2.T
  gu_g, gihist, gamma_i = _sc_gather(users, items, Gu, Gi, ui1)
  xui2, gu_star = _tc_mlp(gu_g, gihist, gamma_i, wa, wb, w2t,
                          b1.reshape(1, K), b2.reshape(1, K))
  return (xui2[:, 0], gu_star, gamma_i)
